# baseline (speedup 1.0000x reference)
import functools

import jax
import jax.numpy as jnp
from jax import lax
from jax.experimental import pallas as pl
from jax.experimental.pallas import tpu as pltpu
from jax.experimental.pallas import tpu_sc as plsc

VOCAB = 1000000
SEQ = 200
HID = 64
BATCH = 4096
EPS = 1e-12

NW = 32
ROWS = BATCH * SEQ
RPW = ROWS // NW
CHUNK = 128
NCH = RPW // CHUNK
SUP = 512
SUBS = SUP // CHUNK
NSUP = RPW // SUP
GROUPS = SUP // 16
KUNROLL = 8


def _rsqrt(v):
    i = plsc.bitcast(v, jnp.int32)
    i = jnp.int32(0x5F3759DF) - (i >> 1)
    y = plsc.bitcast(i, jnp.float32)
    for _ in range(3):
        y = y * (1.5 - 0.5 * v * y * y)
    return y


def _make_emb_kernel():
    mesh = plsc.VectorSubcoreMesh(core_axis_name="c", subcore_axis_name="s")

    @functools.partial(
        pl.kernel,
        mesh=mesh,
        compiler_params=pltpu.CompilerParams(
            needs_layout_passes=False, use_tc_tiling_on_sc=False),
        out_type=jax.ShapeDtypeStruct((ROWS, HID), jnp.float32),
        scratch_types=[
            pltpu.VMEM((NCH, CHUNK), jnp.int32),
            pltpu.VMEM((SUP, HID), jnp.float32),
            pltpu.VMEM((SUP, HID), jnp.float32),
            pltpu.VMEM((SEQ, HID), jnp.float32),
            pltpu.VMEM((HID * 16,), jnp.float32),
            pltpu.VMEM((HID * 16,), jnp.float32),
            pltpu.SemaphoreType.DMA,
            pltpu.SemaphoreType.DMA,
            pltpu.SemaphoreType.DMA,
            pltpu.SemaphoreType.DMA,
        ],
    )
    def emb(state_hbm, table_hbm, pos_hbm, gamma_hbm, beta_hbm, out_hbm,
            idx_v, rows0, rows1, pos_v, gamma_v, beta_v, gs0, gs1, os0, os1):
        rows = (rows0, rows1)
        gsem = (gs0, gs1)
        osem = (os0, os1)
        wid = lax.axis_index("s") * 2 + lax.axis_index("c")
        pltpu.sync_copy(state_hbm.at[wid], idx_v)
        pltpu.sync_copy(pos_hbm, pos_v)
        pltpu.sync_copy(gamma_hbm, gamma_v)
        pltpu.sync_copy(beta_hbm, beta_v)
        base_row = wid * RPW
        lanes = lax.iota(jnp.int32, 16)

        def fire_gather(c, b):
            for j in range(SUBS):
                pltpu.async_copy(
                    table_hbm.at[idx_v.at[c * SUBS + j]],
                    rows[b].at[pl.ds(j * CHUNK, CHUNK)], gsem[b])

        def wait_gather(b):
            pltpu.make_async_copy(
                out_hbm.at[pl.ds(0, SUP)], rows[b], gsem[b]).wait()

        def fire_out(c, b):
            pltpu.async_copy(
                rows[b], out_hbm.at[pl.ds(base_row + c * SUP, SUP)], osem[b])

        def wait_out(b):
            pltpu.make_async_copy(
                rows[b], out_hbm.at[pl.ds(0, SUP)], osem[b]).wait()

        def compute(c, b):
            buf = rows[b]
            g0 = base_row + c * SUP

            def group_body(gi, _):
                lr = lanes + gi * 16
                pvec = jnp.mod(g0 + lr, SEQ)

                def p1(k8, carry):
                    s, s2 = carry
                    for kk in range(KUNROLL):
                        hvec = (lanes + k8 * KUNROLL + kk) & (HID - 1)
                        t = plsc.load_gather(buf, [lr, hvec])
                        p = plsc.load_gather(pos_v, [pvec, hvec])
                        x = t + p
                        plsc.store_scatter(buf, [lr, hvec], x)
                        s = s + x
                        s2 = s2 + x * x
                    return s, s2

                zero = jnp.zeros((16,), jnp.float32)
                s, s2 = lax.fori_loop(0, HID // KUNROLL, p1, (zero, zero))
                mean = s * (1.0 / HID)
                var = s2 * (1.0 / HID) - mean * mean
                rstd = _rsqrt(var + EPS)

                def p2(k8, carry):
                    for kk in range(KUNROLL):
                        k = k8 * KUNROLL + kk
                        hvec = (lanes + k) & (HID - 1)
                        x = plsc.load_gather(buf, [lr, hvec])
                        gam = gamma_v[pl.ds(k * 16, 16)]
                        bet = beta_v[pl.ds(k * 16, 16)]
                        y = (x - mean) * rstd * gam + bet
                        plsc.store_scatter(buf, [lr, hvec], y)
                    return carry

                lax.fori_loop(0, HID // KUNROLL, p2, 0)
                return 0

            lax.fori_loop(0, GROUPS, group_body, 0)

        fire_gather(0, 0)
        fire_gather(1, 1)
        wait_gather(0)
        compute(0, 0)
        fire_out(0, 0)

        def pair_body(i, _):
            c2 = 1 + 2 * i
            wait_gather(1)
            wait_out(0)
            fire_gather(c2 + 1, 0)
            compute(c2, 1)
            fire_out(c2, 1)
            wait_gather(0)
            wait_out(1)
            fire_gather(c2 + 2, 1)
            compute(c2 + 1, 0)
            fire_out(c2 + 1, 0)
            return 0

        lax.fori_loop(0, (NSUP - 2) // 2, pair_body, 0)
        wait_gather(1)
        compute(NSUP - 1, 1)
        fire_out(NSUP - 1, 1)
        wait_out(0)
        wait_out(1)

    return emb


_emb_kernel = _make_emb_kernel()


def kernel(state, token_table, pos_table, ln_gamma, ln_beta):
    state_w = state.reshape(NW, NCH, CHUNK)
    rot = (jnp.arange(HID)[:, None] + jnp.arange(16)[None, :]) % HID
    gamma_rot = ln_gamma[rot].reshape(-1)
    beta_rot = ln_beta[rot].reshape(-1)
    out = _emb_kernel(state_w, token_table, pos_table, gamma_rot, beta_rot)
    return out.reshape(BATCH, SEQ, HID)
